# baseline (speedup 1.0000x reference)
import functools

import jax
import jax.numpy as jnp
from jax import lax
from jax.experimental import pallas as pl
from jax.experimental.pallas import tpu as pltpu
from jax.experimental.pallas import tpu_sc as plsc

D = 64
B = 16384
NC = 2
NS = 16
NW = NC * NS
BPW = B // NW
BLK = 128
NBLK = BPW // BLK
CH = 8
NCH = BLK // CH
EG = 62500
RG = 63


def _make_kernel():
    mesh = plsc.VectorSubcoreMesh(core_axis_name="c", subcore_axis_name="s")

    slab = lambda: pltpu.VMEM((CH, 8, 128), jnp.float32)

    @functools.partial(
        pl.kernel,
        mesh=mesh,
        out_type=jax.ShapeDtypeStruct((B,), jnp.float32),
        compiler_params=pltpu.CompilerParams(needs_layout_passes=False),
        scratch_types=[
            pltpu.VMEM((BLK,), jnp.int32),
            pltpu.VMEM((BLK,), jnp.int32),
            pltpu.VMEM((BLK,), jnp.int32),
            pltpu.VMEM((BLK,), jnp.int32),
            pltpu.VMEM((BLK,), jnp.int32),
            pltpu.VMEM((BLK,), jnp.int32),
            [[slab() for _ in range(6)] for _ in range(2)],
            pltpu.VMEM((BPW * 16,), jnp.float32),
            pltpu.VMEM((BPW,), jnp.float32),
            pltpu.SemaphoreType.DMA,
            pltpu.SemaphoreType.DMA,
        ],
    )
    def complex_score(head, relation, tail, ent_r, ent_i, rel_r, rel_i,
                      out, ihv, irv, itv, gh, gr, gt,
                      bufs, stage, out_v, sem0, sem1):
        wid = lax.axis_index("s") * NC + lax.axis_index("c")
        base = wid * BPW
        sems = (sem0, sem1)
        lane16 = lax.iota(jnp.int32, 16) * 16

        def srcs(c, slot):
            del slot
            s = pl.ds(c * CH, CH)
            return (
                (ent_r.at[gh.at[s]], 0), (ent_i.at[gh.at[s]], 1),
                (ent_r.at[gt.at[s]], 2), (ent_i.at[gt.at[s]], 3),
                (rel_r.at[gr.at[s]], 4), (rel_i.at[gr.at[s]], 5),
            )

        def fire(c, slot):
            for src, t in srcs(c, slot):
                pltpu.async_copy(src, bufs[slot][t], sems[slot])

        def drain(c, slot):
            for src, t in srcs(c, slot):
                pltpu.make_async_copy(src, bufs[slot][t], sems[slot]).wait()

        def compute(blk, c, slot, idsh, idsr, idst, lo):
            for u in range(CH):
                i = c * CH + u
                rh = idsh[lo + u]
                rt = idst[lo + u]
                rq = idsr[lo + u]
                sh, hh = (rh >> 1) & 7, (rh & 1) * 64
                st, ht = (rt >> 1) & 7, (rt & 1) * 64
                sq, hq = (rq >> 1) & 7, (rq & 1) * 64
                acc = jnp.zeros((16,), jnp.float32)
                for k in range(D // 16):
                    hrv = bufs[slot][0][u, sh, pl.ds(hh + k * 16, 16)]
                    hiv = bufs[slot][1][u, sh, pl.ds(hh + k * 16, 16)]
                    trv = bufs[slot][2][u, st, pl.ds(ht + k * 16, 16)]
                    tiv = bufs[slot][3][u, st, pl.ds(ht + k * 16, 16)]
                    rrv = bufs[slot][4][u, sq, pl.ds(hq + k * 16, 16)]
                    riv = bufs[slot][5][u, sq, pl.ds(hq + k * 16, 16)]
                    a = hrv * trv - hiv * tiv
                    bb = hrv * tiv + hiv * trv
                    acc = acc + rrv * a + riv * bb
                stage[pl.ds((blk * BLK + i) * 16, 16)] = acc

        def block(blk, _):
            off = base + blk * BLK
            pltpu.sync_copy(head.at[pl.ds(off, BLK)], ihv)
            pltpu.sync_copy(relation.at[pl.ds(off, BLK)], irv)
            pltpu.sync_copy(tail.at[pl.ds(off, BLK)], itv)
            for s8 in range(BLK // 16):
                s = pl.ds(s8 * 16, 16)
                gh[s] = ihv[s] >> 4
                gr[s] = irv[s] >> 4
                gt[s] = itv[s] >> 4
            fire(0, 0)

            def step(t, _):
                c = t * 2
                s16 = pl.ds(t * 16, 16)
                idsh = ihv[s16]
                idsr = irv[s16]
                idst = itv[s16]
                fire(c + 1, 1)
                drain(c, 0)
                compute(blk, c, 0, idsh, idsr, idst, 0)

                @pl.when(c + 2 < NCH)
                def _():
                    fire(c + 2, 0)

                drain(c + 1, 1)
                compute(blk, c + 1, 1, idsh, idsr, idst, CH)
                return 0

            lax.fori_loop(0, NCH // 2, step, 0)
            return 0

        lax.fori_loop(0, NBLK, block, 0)

        def group(g, _):
            gbase = g * 256
            tot = jnp.zeros((16,), jnp.float32)
            for j in range(16):
                tot = tot + plsc.load_gather(stage, [gbase + lane16 + j])
            out_v[pl.ds(g * 16, 16)] = tot
            return 0

        lax.fori_loop(0, BPW // 16, group, 0)
        pltpu.sync_copy(out_v, out.at[pl.ds(base, BPW)])

    return complex_score


_KERNEL = _make_kernel()


def kernel(head, relation, tail, entity_real, entity_imag,
           relation_real, relation_imag):
    ent_r = jnp.reshape(entity_real, (EG, 8, 128))
    ent_i = jnp.reshape(entity_imag, (EG, 8, 128))
    pad = ((0, 8), (0, 0))
    rel_r = jnp.reshape(jnp.pad(relation_real, pad), (RG, 8, 128))
    rel_i = jnp.reshape(jnp.pad(relation_imag, pad), (RG, 8, 128))
    return _KERNEL(head, relation, tail, ent_r, ent_i, rel_r, rel_i)
